# baseline (speedup 1.0000x reference)
import jax
import jax.numpy as jnp
from jax.experimental import pallas as pl


_ROWS_PER_BLOCK = 2000


def _copy_block(src_ref, dst_ref):
    dst_ref[...] = src_ref[...]


def _copy(x):
    n, d = x.shape
    grid = (n // _ROWS_PER_BLOCK,)
    return pl.pallas_call(
        _copy_block,
        grid=grid,
        in_specs=[pl.BlockSpec((_ROWS_PER_BLOCK, d), lambda i: (i, 0))],
        out_specs=pl.BlockSpec((_ROWS_PER_BLOCK, d), lambda i: (i, 0)),
        out_shape=jax.ShapeDtypeStruct((n, d), x.dtype),
    )(x)


def kernel(embed_paper, embed_author, embed_field):
    return (_copy(embed_paper), _copy(embed_author), _copy(embed_field))
